# baseline (speedup 1.0000x reference)
import functools

import jax
import jax.numpy as jnp
from jax import lax
from jax.experimental import pallas as pl
from jax.experimental.pallas import tpu as pltpu
from jax.experimental.pallas import tpu_sc as plsc

N_FIELDS = 26
VOCAB = 1000
EMB_DIM = 1000
BATCH = 1024
ROWS = BATCH * N_FIELDS
NC, NS, L = 2, 16, 16
NW = NC * NS
ROWS_PER_W = ROWS // NW
CHUNK = 64
NCHUNK = ROWS_PER_W // CHUNK


def _make_gather():
    mesh = plsc.VectorSubcoreMesh(core_axis_name="c", subcore_axis_name="s")

    @functools.partial(
        pl.kernel,
        mesh=mesh,
        out_type=jax.ShapeDtypeStruct((ROWS, EMB_DIM), jnp.float32),
        scratch_types=[
            pltpu.VMEM((NCHUNK, CHUNK), jnp.int32),
            pltpu.VMEM((CHUNK, EMB_DIM), jnp.float32),
            pltpu.SemaphoreType.DMA,
        ],
        compiler_params=pltpu.CompilerParams(use_tc_tiling_on_sc=False,
                                             needs_layout_passes=False),
    )
    def gather_kernel(table, idx_hbm, out, idx_v, buf, sem):
        wid = lax.axis_index("s") * NC + lax.axis_index("c")
        pltpu.sync_copy(idx_hbm.at[wid], idx_v)
        zeros16 = jnp.zeros((L,), jnp.float32)
        for c in range(NCHUNK):
            pltpu.async_copy(table.at[idx_v.at[c]], buf, sem).wait()
            for g in range(CHUNK // L):
                v = idx_v[c, pl.ds(g * L, L)]
                remv = lax.rem(v, jnp.full((L,), VOCAB, jnp.int32))
                min_rem = jnp.min(remv)

                @pl.when(min_rem == 0)
                def _zero_rows(g=g, remv=remv):
                    pad = remv == jnp.zeros((L,), jnp.int32)
                    rows = g * L + lax.broadcasted_iota(jnp.int32, (L,), 0)

                    def body(col, carry):
                        cols = jnp.full((L,), col, jnp.int32)
                        plsc.store_scatter(buf, [rows, cols], zeros16,
                                           mask=pad)
                        return carry

                    lax.fori_loop(0, EMB_DIM, body, 0)

            base = wid * ROWS_PER_W + c * CHUNK
            pltpu.sync_copy(buf, out.at[pl.ds(base, CHUNK)])

    return gather_kernel


_gather = _make_gather()


def kernel(x, tables):
    table_flat = tables.reshape(N_FIELDS * VOCAB, EMB_DIM)
    offs = (jnp.arange(N_FIELDS, dtype=jnp.int32) * VOCAB)[None, :]
    gidx = (x + offs).reshape(NW, NCHUNK, CHUNK)
    out = _gather(table_flat, gidx)
    return out.reshape(BATCH, N_FIELDS * EMB_DIM)


# speedup vs baseline: 1.0370x; 1.0370x over previous
import functools

import jax
import jax.numpy as jnp
from jax import lax
from jax.experimental import pallas as pl
from jax.experimental.pallas import tpu as pltpu
from jax.experimental.pallas import tpu_sc as plsc

N_FIELDS = 26
VOCAB = 1000
EMB_DIM = 1000
BATCH = 1024
ROWS = BATCH * N_FIELDS
NC, NS, L = 2, 16, 16
NW = NC * NS
ROWS_PER_W = ROWS // NW
CHUNK = 64
NCHUNK = ROWS_PER_W // CHUNK


def _make_gather():
    mesh = plsc.VectorSubcoreMesh(core_axis_name="c", subcore_axis_name="s")

    @functools.partial(
        pl.kernel,
        mesh=mesh,
        out_type=jax.ShapeDtypeStruct((ROWS, EMB_DIM), jnp.float32),
        scratch_types=[
            pltpu.VMEM((NCHUNK, CHUNK), jnp.int32),
            pltpu.VMEM((CHUNK, EMB_DIM), jnp.float32),
            pltpu.VMEM((CHUNK, EMB_DIM), jnp.float32),
            pltpu.SemaphoreType.DMA,
            pltpu.SemaphoreType.DMA,
            pltpu.SemaphoreType.DMA,
            pltpu.SemaphoreType.DMA,
        ],
        compiler_params=pltpu.CompilerParams(use_tc_tiling_on_sc=False,
                                             needs_layout_passes=False),
    )
    def gather_kernel(table, idx_hbm, out, idx_v,
                      buf0, buf1, gsem0, gsem1, ssem0, ssem1):
        wid = lax.axis_index("s") * NC + lax.axis_index("c")
        pltpu.sync_copy(idx_hbm.at[wid], idx_v)
        zeros16 = jnp.zeros((L,), jnp.float32)
        bufs = (buf0, buf1)
        gsems = (gsem0, gsem1)
        ssems = (ssem0, ssem1)

        def issue_gather(c, b):
            return pltpu.async_copy(table.at[idx_v.at[c]], bufs[b], gsems[b])

        def mask_chunk(c, b):
            for g in range(CHUNK // L):
                v = idx_v[c, pl.ds(g * L, L)]
                remv = lax.rem(v, jnp.full((L,), VOCAB, jnp.int32))
                min_rem = jnp.min(remv)

                @pl.when(min_rem == 0)
                def _zero_rows(g=g, remv=remv, b=b):
                    pad = remv == jnp.zeros((L,), jnp.int32)
                    rows = g * L + lax.broadcasted_iota(jnp.int32, (L,), 0)

                    def body(col, carry):
                        cols = jnp.full((L,), col, jnp.int32)
                        plsc.store_scatter(bufs[b], [rows, cols], zeros16,
                                           mask=pad)
                        return carry

                    lax.fori_loop(0, EMB_DIM, body, 0)

        gcopies = {0: issue_gather(0, 0), 1: issue_gather(1, 1)}
        for c in range(NCHUNK):
            b = c % 2
            gcopies[c].wait()
            mask_chunk(c, b)
            base = wid * ROWS_PER_W + c * CHUNK
            scp = pltpu.async_copy(bufs[b], out.at[pl.ds(base, CHUNK)],
                                   ssems[b])
            scp.wait()
            if c + 2 < NCHUNK:
                gcopies[c + 2] = issue_gather(c + 2, b)

    return gather_kernel


_gather = _make_gather()


def kernel(x, tables):
    table_flat = tables.reshape(N_FIELDS * VOCAB, EMB_DIM)
    offs = (jnp.arange(N_FIELDS, dtype=jnp.int32) * VOCAB)[None, :]
    gidx = (x + offs).reshape(NW, NCHUNK, CHUNK)
    out = _gather(table_flat, gidx)
    return out.reshape(BATCH, N_FIELDS * EMB_DIM)
